# baseline (speedup 1.0000x reference)
import functools

import jax
import jax.numpy as jnp
from jax import lax
from jax.experimental import pallas as pl
from jax.experimental.pallas import tpu as pltpu
from jax.experimental.pallas import tpu_sc as plsc

_NC = 2
_NS = 16
_NW = _NC * _NS
_L = 16
_LANES = 128

_CHUNK = 2048
_G_ELEMS = 40960


def _take(tab, idx):
    return tab.at[idx].get(mode="promise_in_bounds")


@functools.lru_cache(maxsize=None)
def _build_sc_kernel(n: int, k: int):
    assert k == 8, "kernel is specialized to K == 8 mixture components"
    per_w = n // _NW
    assert per_w * _NW == n
    g_elems = _G_ELEMS if per_w > _G_ELEMS else 0
    lin = per_w - g_elems
    chunk = min(_CHUNK, lin)
    n_ch = lin // chunk
    assert n_ch * chunk == lin
    assert chunk % _LANES == 0 and n % _LANES == 0
    assert g_elems % _LANES == 0
    tile = k * _LANES

    mesh = plsc.VectorSubcoreMesh(
        core_axis_name="c", subcore_axis_name="s", num_cores=_NC, num_subcores=_NS
    )

    scratch = [
        pltpu.VMEM((max(g_elems, 1),), jnp.int32),
        pltpu.VMEM((max(g_elems, 1),), jnp.float32),
        pltpu.VMEM((chunk * k,), jnp.float32),
        pltpu.VMEM((chunk * k,), jnp.float32),
        pltpu.VMEM((chunk,), jnp.int32),
        pltpu.VMEM((chunk,), jnp.int32),
        pltpu.VMEM((chunk,), jnp.float32),
        pltpu.VMEM((chunk,), jnp.float32),
        pltpu.VMEM((2 * k,), jnp.float32),
        pltpu.SemaphoreType.DMA,
        pltpu.SemaphoreType.DMA,
        pltpu.SemaphoreType.DMA,
        pltpu.SemaphoreType.DMA,
        pltpu.SemaphoreType.DMA,
    ]

    @functools.partial(
        pl.kernel,
        mesh=mesh,
        compiler_params=pltpu.CompilerParams(needs_layout_passes=False),
        out_type=jax.ShapeDtypeStruct((n,), jnp.float32),
        scratch_types=scratch,
    )
    def sc_kernel(eps_hbm, ms_hbm, tab_hbm, out_hbm,
                  gi, gg, eb0, eb1, mb0, mb1, ob0, ob1, tab_v,
                  gsem, ls0, ls1, ss0, ss1):
        eb = (eb0, eb1)
        mb = (mb0, mb1)
        ob = (ob0, ob1)
        lsem = (ls0, ls1)
        ssem = (ss0, ss1)

        wid = lax.axis_index("s") * _NC + lax.axis_index("c")
        base = wid * per_w

        pltpu.sync_copy(tab_hbm, tab_v)
        tab = tab_v[...]

        iota = lax.iota(jnp.int32, _L)

        gather_d = None
        if g_elems:
            pltpu.sync_copy(ms_hbm.at[pl.ds(base, g_elems)], gi)

            @plsc.parallel_loop(0, g_elems, _L, unroll=8)
            def gp1(j):
                sl = pl.ds(j, _L)
                i0 = base + j
                s = (i0 // _LANES) * tile + (i0 % _LANES)
                gi[sl] = lax.shift_left(gi[sl], 7) + (s + iota)

            gather_d = pltpu.async_copy(eps_hbm.at[gi], gg, gsem)

        lbase = base + g_elems

        def start_loads(c, b):
            off = lbase + c * chunk
            d1 = pltpu.async_copy(
                eps_hbm.at[pl.ds(off * k, chunk * k)], eb[b], lsem[b]
            )
            d2 = pltpu.async_copy(ms_hbm.at[pl.ds(off, chunk)], mb[b], lsem[b])
            return (d1, d2)

        def compute(b):
            @plsc.parallel_loop(0, chunk, _L, unroll=8)
            def p(j):
                sl = pl.ds(j, _L)
                m = mb[b][sl]
                lo = _take(tab, m)
                sc = _take(tab, m + k)
                s = (j // _LANES) * tile + (j % _LANES)
                lidx = lax.shift_left(m, 7) + (s + iota)
                g = plsc.load_gather(eb[b], [lidx])
                ob[b][sl] = lo + sc * g

        def start_store(c, b):
            off = lbase + c * chunk
            return pltpu.async_copy(ob[b], out_hbm.at[pl.ds(off, chunk)], ssem[b])

        load_d = [None, None]
        store_d = [None, None]
        if n_ch > 0:
            load_d[0] = start_loads(0, 0)
        if n_ch > 1:
            load_d[1] = start_loads(1, 1)
        for c in range(n_ch):
            b = c & 1
            for d in load_d[b]:
                d.wait()
            if store_d[b] is not None:
                store_d[b].wait()
                store_d[b] = None
            compute(b)
            store_d[b] = start_store(c, b)
            if c + 2 < n_ch:
                load_d[b] = start_loads(c + 2, b)
        for b in range(2):
            if store_d[b] is not None:
                store_d[b].wait()

        if g_elems:
            gather_d.wait()

            @plsc.parallel_loop(0, g_elems, _L, unroll=8)
            def gp2(j):
                sl = pl.ds(j, _L)
                fl = gi[sl]
                g = gg[sl]
                m = jnp.bitwise_and(lax.shift_right_logical(fl, 7), k - 1)
                lo = _take(tab, m)
                sc = _take(tab, m + k)
                gg[sl] = lo + sc * g

            pltpu.sync_copy(gg, out_hbm.at[pl.ds(base, g_elems)])

    return sc_kernel


def kernel(eps, ms, location, scale):
    n, k = eps.shape
    sc_kernel = _build_sc_kernel(n, k)
    eps_native = (
        eps.reshape(n // _LANES, _LANES, k).transpose(0, 2, 1).reshape(n * k)
    )
    tab = jnp.concatenate(
        [location.astype(jnp.float32), scale.astype(jnp.float32)]
    )
    return sc_kernel(eps_native, ms.astype(jnp.int32), tab)


# speedup vs baseline: 1.3737x; 1.3737x over previous
import functools

import jax
import jax.numpy as jnp
from jax import lax
from jax.experimental import pallas as pl
from jax.experimental.pallas import tpu as pltpu
from jax.experimental.pallas import tpu_sc as plsc

_NC = 2
_NS = 16
_NW = _NC * _NS
_L = 16
_LANES = 128

_CHUNK = 4096


def _take(tab, idx):
    return tab.at[idx].get(mode="promise_in_bounds")


@functools.lru_cache(maxsize=None)
def _build_sc_kernel(n: int, k: int):
    assert k == 8, "kernel is specialized to K == 8 mixture components"
    per_w = n // _NW
    assert per_w * _NW == n
    chunk = min(_CHUNK, per_w)
    n_ch = per_w // chunk
    assert n_ch * chunk == per_w
    assert chunk % _LANES == 0 and n % _LANES == 0
    tile = k * _LANES

    mesh = plsc.VectorSubcoreMesh(
        core_axis_name="c", subcore_axis_name="s", num_cores=_NC, num_subcores=_NS
    )

    @functools.partial(
        pl.kernel,
        mesh=mesh,
        compiler_params=pltpu.CompilerParams(needs_layout_passes=False),
        out_type=jax.ShapeDtypeStruct((n,), jnp.float32),
        scratch_types=[
            pltpu.VMEM((chunk * k,), jnp.float32),
            pltpu.VMEM((chunk * k,), jnp.float32),
            pltpu.VMEM((chunk * k,), jnp.float32),
            pltpu.VMEM((chunk,), jnp.int32),
            pltpu.VMEM((chunk,), jnp.int32),
            pltpu.VMEM((chunk,), jnp.int32),
            pltpu.VMEM((chunk,), jnp.float32),
            pltpu.VMEM((chunk,), jnp.float32),
            pltpu.VMEM((chunk,), jnp.float32),
            pltpu.VMEM((2 * k,), jnp.float32),
            pltpu.SemaphoreType.DMA,
            pltpu.SemaphoreType.DMA,
            pltpu.SemaphoreType.DMA,
            pltpu.SemaphoreType.DMA,
            pltpu.SemaphoreType.DMA,
            pltpu.SemaphoreType.DMA,
        ],
    )
    def sc_kernel(eps_hbm, ms_hbm, tab_hbm, out_hbm,
                  eb0, eb1, eb2, mb0, mb1, mb2, ob0, ob1, ob2, tab_v,
                  ls0, ls1, ls2, ss0, ss1, ss2):
        eb = (eb0, eb1, eb2)
        mb = (mb0, mb1, mb2)
        ob = (ob0, ob1, ob2)
        lsem = (ls0, ls1, ls2)
        ssem = (ss0, ss1, ss2)

        wid = lax.axis_index("s") * _NC + lax.axis_index("c")
        base = wid * per_w

        pltpu.sync_copy(tab_hbm, tab_v)
        tab = tab_v[...]

        iota = lax.iota(jnp.int32, _L)

        def start_loads(c, b):
            off = base + c * chunk
            d1 = pltpu.async_copy(
                eps_hbm.at[pl.ds(off * k, chunk * k)], eb[b], lsem[b]
            )
            d2 = pltpu.async_copy(ms_hbm.at[pl.ds(off, chunk)], mb[b], lsem[b])
            return (d1, d2)

        def compute(b):
            @plsc.parallel_loop(0, chunk, _L, unroll=8)
            def p(j):
                sl = pl.ds(j, _L)
                m = mb[b][sl]
                lo = _take(tab, m)
                sc = _take(tab, m + k)
                s = (j // _LANES) * tile + (j % _LANES)
                lidx = lax.shift_left(m, 7) + (s + iota)
                g = plsc.load_gather(eb[b], [lidx])
                ob[b][sl] = lo + sc * g

        def start_store(c, b):
            off = base + c * chunk
            return pltpu.async_copy(ob[b], out_hbm.at[pl.ds(off, chunk)], ssem[b])

        nb = 3
        load_d = [None] * nb
        store_d = [None] * nb
        for c0 in range(min(nb, n_ch)):
            load_d[c0] = start_loads(c0, c0)
        for c in range(n_ch):
            b = c % nb
            for d in load_d[b]:
                d.wait()
            if store_d[b] is not None:
                store_d[b].wait()
                store_d[b] = None
            compute(b)
            store_d[b] = start_store(c, b)
            if c + nb < n_ch:
                load_d[b] = start_loads(c + nb, b)
        for b in range(nb):
            if store_d[b] is not None:
                store_d[b].wait()

    return sc_kernel


def kernel(eps, ms, location, scale):
    n, k = eps.shape
    sc_kernel = _build_sc_kernel(n, k)
    eps_native = (
        eps.reshape(n // _LANES, _LANES, k).transpose(0, 2, 1).reshape(n * k)
    )
    tab = jnp.concatenate(
        [location.astype(jnp.float32), scale.astype(jnp.float32)]
    )
    return sc_kernel(eps_native, ms.astype(jnp.int32), tab)
